# Initial kernel scaffold; baseline (speedup 1.0000x reference)
import jax
import jax.numpy as jnp
from jax.experimental import pallas as pl


def kernel(x, W, b):
    raise NotImplementedError("write your pallas kernel here")


# speedup vs baseline: 2.0453x; 2.0453x over previous
import functools

import jax
import jax.numpy as jnp
from jax import lax
from jax.experimental import pallas as pl
from jax.experimental.pallas import tpu as pltpu

_INPUT_SIZE = 768
_NUM_EXPERTS = 64
_BT = 1024


def _gate_body(x_ref, wt_ref, b_ref, s_ref, i_ref):
    scores = jnp.dot(x_ref[...], wt_ref[...],
                     preferred_element_type=jnp.float32) + b_ref[...]
    col = lax.broadcasted_iota(jnp.int32, scores.shape, 1)
    m1 = jnp.max(scores, axis=1, keepdims=True)
    i1 = jnp.min(jnp.where(scores == m1, col, _NUM_EXPERTS),
                 axis=1, keepdims=True)
    masked = jnp.where(col == i1, -jnp.inf, scores)
    m2 = jnp.max(masked, axis=1, keepdims=True)
    i2 = jnp.min(jnp.where(masked == m2, col, _NUM_EXPERTS),
                 axis=1, keepdims=True)
    e = jnp.exp(m2 - m1)
    denom = 1.0 + e
    s_ref[:, 0:1] = 1.0 / denom
    s_ref[:, 1:2] = e / denom
    i_ref[:, 0:1] = i1
    i_ref[:, 1:2] = i2


def kernel(x, W, b):
    n_tokens = x.shape[0]
    wt = W.T
    b2 = b.reshape(1, _NUM_EXPERTS)
    grid = (n_tokens // _BT,)
    out_s, out_i = pl.pallas_call(
        _gate_body,
        grid=grid,
        in_specs=[
            pl.BlockSpec((_BT, _INPUT_SIZE), lambda i: (i, 0)),
            pl.BlockSpec((_INPUT_SIZE, _NUM_EXPERTS), lambda i: (0, 0)),
            pl.BlockSpec((1, _NUM_EXPERTS), lambda i: (0, 0)),
        ],
        out_specs=[
            pl.BlockSpec((_BT, 2), lambda i: (i, 0)),
            pl.BlockSpec((_BT, 2), lambda i: (i, 0)),
        ],
        out_shape=[
            jax.ShapeDtypeStruct((n_tokens, 2), jnp.float32),
            jax.ShapeDtypeStruct((n_tokens, 2), jnp.int32),
        ],
        compiler_params=pltpu.CompilerParams(
            dimension_semantics=("arbitrary",),
        ),
    )(x, wt, b2)
    return out_s, out_i


# speedup vs baseline: 2.3414x; 1.1448x over previous
import functools

import jax
import jax.numpy as jnp
from jax import lax
from jax.experimental import pallas as pl
from jax.experimental.pallas import tpu as pltpu

_INPUT_SIZE = 768
_NUM_EXPERTS = 64
_BT = 2048


def _gate_body(x_ref, wt_ref, b_ref, s_ref, i_ref):
    scores = jnp.dot(x_ref[...], wt_ref[...],
                     preferred_element_type=jnp.float32) + b_ref[...]
    col = lax.broadcasted_iota(jnp.int32, scores.shape, 1)
    m1 = jnp.max(scores, axis=1, keepdims=True)
    i1 = jnp.min(jnp.where(scores == m1, col, _NUM_EXPERTS),
                 axis=1, keepdims=True)
    masked = jnp.where(col == i1, -jnp.inf, scores)
    m2 = jnp.max(masked, axis=1, keepdims=True)
    i2 = jnp.min(jnp.where(masked == m2, col, _NUM_EXPERTS),
                 axis=1, keepdims=True)
    e = jnp.exp(m2 - m1)
    denom = 1.0 + e
    s_ref[:, 0:1] = 1.0 / denom
    s_ref[:, 1:2] = e / denom
    i_ref[:, 0:1] = i1
    i_ref[:, 1:2] = i2


def kernel(x, W, b):
    n_tokens = x.shape[0]
    wt = W.T
    b2 = b.reshape(1, _NUM_EXPERTS)
    grid = (n_tokens // _BT,)
    out_s, out_i = pl.pallas_call(
        _gate_body,
        grid=grid,
        in_specs=[
            pl.BlockSpec((_BT, _INPUT_SIZE), lambda i: (i, 0)),
            pl.BlockSpec((_INPUT_SIZE, _NUM_EXPERTS), lambda i: (0, 0)),
            pl.BlockSpec((1, _NUM_EXPERTS), lambda i: (0, 0)),
        ],
        out_specs=[
            pl.BlockSpec((_BT, 2), lambda i: (i, 0)),
            pl.BlockSpec((_BT, 2), lambda i: (i, 0)),
        ],
        out_shape=[
            jax.ShapeDtypeStruct((n_tokens, 2), jnp.float32),
            jax.ShapeDtypeStruct((n_tokens, 2), jnp.int32),
        ],
        compiler_params=pltpu.CompilerParams(
            dimension_semantics=("arbitrary",),
        ),
    )(x, wt, b2)
    return out_s, out_i


# speedup vs baseline: 2.5119x; 1.0728x over previous
import functools

import jax
import jax.numpy as jnp
from jax import lax
from jax.experimental import pallas as pl
from jax.experimental.pallas import tpu as pltpu

_INPUT_SIZE = 768
_NUM_EXPERTS = 64
_BT = 4096


def _gate_body(x_ref, wt_ref, b_ref, s_ref, i_ref):
    scores = jnp.dot(x_ref[...], wt_ref[...],
                     preferred_element_type=jnp.float32) + b_ref[...]
    col = lax.broadcasted_iota(jnp.int32, scores.shape, 1)
    m1 = jnp.max(scores, axis=1, keepdims=True)
    i1 = jnp.min(jnp.where(scores == m1, col, _NUM_EXPERTS),
                 axis=1, keepdims=True)
    masked = jnp.where(col == i1, -jnp.inf, scores)
    m2 = jnp.max(masked, axis=1, keepdims=True)
    i2 = jnp.min(jnp.where(masked == m2, col, _NUM_EXPERTS),
                 axis=1, keepdims=True)
    e = jnp.exp(m2 - m1)
    denom = 1.0 + e
    s_ref[:, 0:1] = 1.0 / denom
    s_ref[:, 1:2] = e / denom
    i_ref[:, 0:1] = i1
    i_ref[:, 1:2] = i2


def kernel(x, W, b):
    n_tokens = x.shape[0]
    wt = W.T
    b2 = b.reshape(1, _NUM_EXPERTS)
    grid = (n_tokens // _BT,)
    out_s, out_i = pl.pallas_call(
        _gate_body,
        grid=grid,
        in_specs=[
            pl.BlockSpec((_BT, _INPUT_SIZE), lambda i: (i, 0)),
            pl.BlockSpec((_INPUT_SIZE, _NUM_EXPERTS), lambda i: (0, 0)),
            pl.BlockSpec((1, _NUM_EXPERTS), lambda i: (0, 0)),
        ],
        out_specs=[
            pl.BlockSpec((_BT, 2), lambda i: (i, 0)),
            pl.BlockSpec((_BT, 2), lambda i: (i, 0)),
        ],
        out_shape=[
            jax.ShapeDtypeStruct((n_tokens, 2), jnp.float32),
            jax.ShapeDtypeStruct((n_tokens, 2), jnp.int32),
        ],
        compiler_params=pltpu.CompilerParams(
            dimension_semantics=("arbitrary",),
        ),
    )(x, wt, b2)
    return out_s, out_i
